# baseline (speedup 1.0000x reference)
import functools

import jax
import jax.numpy as jnp
from jax import lax
from jax.experimental import pallas as pl
from jax.experimental.pallas import tpu as pltpu

E = 8
K = 2
T = 2048
D = 2048
H = 2048

BT = 128
NB = (T * K) // BT + E
P = NB * BT
NH = 4
BH = H // NH


def _ffn_block(gid_ref, x_ref, gate_ref, lin_ref, down_ref, out_ref,
               gate_bf, lin_bf, down_bf):
    b = pl.program_id(1)
    prev = gid_ref[jnp.maximum(b - 1, 0)]
    changed = jnp.logical_or(b == 0, gid_ref[b] != prev)

    @pl.when(changed)
    def _repack():
        gate_bf[...] = gate_ref[0].astype(jnp.bfloat16)
        lin_bf[...] = lin_ref[0].astype(jnp.bfloat16)
        down_bf[...] = down_ref[0].astype(jnp.bfloat16)

    x = x_ref[...]
    g = lax.dot_general(x, gate_bf[...], (((1,), (1,)), ((), ())),
                        preferred_element_type=jnp.float32)
    l = lax.dot_general(x, lin_bf[...], (((1,), (1,)), ((), ())),
                        preferred_element_type=jnp.float32)
    act = ((g * jax.nn.sigmoid(g)) * l).astype(jnp.bfloat16)
    out_ref[0] = lax.dot_general(act, down_bf[...], (((1,), (1,)), ((), ())),
                                 preferred_element_type=jnp.float32
                                 ).astype(jnp.bfloat16)


def _grouped_ffn(x_sorted, gid, up_weight, down_weight):
    spec = pltpu.PrefetchScalarGridSpec(
        num_scalar_prefetch=1,
        grid=(NH, NB),
        in_specs=[
            pl.BlockSpec((BT, D), lambda h, b, gid: (b, 0)),
            pl.BlockSpec((1, BH, D), lambda h, b, gid: (gid[b], h, 0)),
            pl.BlockSpec((1, BH, D), lambda h, b, gid: (gid[b], h + NH, 0)),
            pl.BlockSpec((1, D, BH), lambda h, b, gid: (gid[b], 0, h)),
        ],
        out_specs=pl.BlockSpec((1, BT, D), lambda h, b, gid: (h, b, 0)),
        scratch_shapes=[
            pltpu.VMEM((BH, D), jnp.bfloat16),
            pltpu.VMEM((BH, D), jnp.bfloat16),
            pltpu.VMEM((D, BH), jnp.bfloat16),
        ],
    )
    return pl.pallas_call(
        _ffn_block,
        grid_spec=spec,
        out_shape=jax.ShapeDtypeStruct((NH, P, D), jnp.bfloat16),
    )(gid, x_sorted, up_weight, up_weight, down_weight)


def kernel(hidden_states, topk_weights, topk_ids, up_weight, down_weight):
    ids = topk_ids.T.reshape(-1)
    counts = jnp.bincount(ids, length=E)
    starts = jnp.concatenate([jnp.zeros((1,), jnp.int32),
                              jnp.cumsum(counts)[:-1].astype(jnp.int32)])
    nb = (counts + BT - 1) // BT
    cum_nb = jnp.cumsum(nb)
    aligned_off = BT * jnp.concatenate(
        [jnp.zeros((1,), jnp.int32), cum_nb[:-1].astype(jnp.int32)])
    order = jnp.argsort(ids, stable=True)
    inv = jnp.argsort(order)
    rank_in_e = inv - starts[ids]
    pos = aligned_off[ids] + rank_in_e

    b_iota = jnp.arange(NB)
    gid = jnp.minimum(
        jnp.sum(b_iota[:, None] >= cum_nb[None, :], axis=1), E - 1
    ).astype(jnp.int32)

    tok = jnp.arange(T * K) % T
    x_sorted = (jnp.zeros((P, D), jnp.bfloat16)
                .at[pos].set(hidden_states[tok].astype(jnp.bfloat16)))

    d = _grouped_ffn(x_sorted, gid, up_weight, down_weight)

    pos2 = pos.reshape(K, T)
    out = jnp.zeros((T, D), jnp.float32)
    for k in range(K):
        rows = jnp.zeros((T, D), jnp.float32)
        for h in range(NH):
            rows = rows + d[h][pos2[k]].astype(jnp.float32)
        out = out + topk_weights[:, k][:, None] * rows
    return out


# speedup vs baseline: 1.2898x; 1.2898x over previous
import functools

import jax
import jax.numpy as jnp
from jax import lax
from jax.experimental import pallas as pl
from jax.experimental.pallas import tpu as pltpu

E = 8
K = 2
T = 2048
D = 2048
H = 2048

BT = 128
NB = (T * K) // BT + E
P = NB * BT
NH = 2
BH = H // NH


def _ffn_block(gid_ref, x_ref, gate_ref, lin_ref, down_ref, out_ref):
    x = x_ref[...]
    g = lax.dot_general(x, gate_ref[0], (((1,), (1,)), ((), ())),
                        preferred_element_type=jnp.float32)
    l = lax.dot_general(x, lin_ref[0], (((1,), (1,)), ((), ())),
                        preferred_element_type=jnp.float32)
    act = (g * jax.nn.sigmoid(g)) * l
    out_ref[0] = lax.dot_general(act, down_ref[0], (((1,), (1,)), ((), ())),
                                 preferred_element_type=jnp.float32
                                 ).astype(jnp.bfloat16)


def _grouped_ffn(x_sorted, gid, up_weight, down_weight):
    spec = pltpu.PrefetchScalarGridSpec(
        num_scalar_prefetch=1,
        grid=(NH, NB),
        in_specs=[
            pl.BlockSpec((BT, D), lambda h, b, gid: (b, 0)),
            pl.BlockSpec((1, BH, D), lambda h, b, gid: (gid[b], h, 0)),
            pl.BlockSpec((1, BH, D), lambda h, b, gid: (gid[b], h + NH, 0)),
            pl.BlockSpec((1, D, BH), lambda h, b, gid: (gid[b], 0, h)),
        ],
        out_specs=pl.BlockSpec((1, BT, D), lambda h, b, gid: (h, b, 0)),
    )
    return pl.pallas_call(
        _ffn_block,
        grid_spec=spec,
        out_shape=jax.ShapeDtypeStruct((NH, P, D), jnp.bfloat16),
    )(gid, x_sorted, up_weight, up_weight, down_weight)


def kernel(hidden_states, topk_weights, topk_ids, up_weight, down_weight):
    ids = topk_ids.T.reshape(-1)
    counts = jnp.bincount(ids, length=E)
    starts = jnp.concatenate([jnp.zeros((1,), jnp.int32),
                              jnp.cumsum(counts)[:-1].astype(jnp.int32)])
    nb = (counts + BT - 1) // BT
    cum_nb = jnp.cumsum(nb)
    aligned_off = BT * jnp.concatenate(
        [jnp.zeros((1,), jnp.int32), cum_nb[:-1].astype(jnp.int32)])
    order = jnp.argsort(ids, stable=True)
    inv = jnp.argsort(order)
    rank_in_e = inv - starts[ids]
    pos = aligned_off[ids] + rank_in_e

    b_iota = jnp.arange(NB)
    gid = jnp.minimum(
        jnp.sum(b_iota[:, None] >= cum_nb[None, :], axis=1), E - 1
    ).astype(jnp.int32)

    tok = jnp.arange(T * K) % T
    x_sorted = jnp.zeros((P, D), jnp.float32).at[pos].set(hidden_states[tok])

    d = _grouped_ffn(x_sorted, gid, up_weight, down_weight)

    pos2 = pos.reshape(K, T)
    out = jnp.zeros((T, D), jnp.float32)
    for k in range(K):
        rows = jnp.zeros((T, D), jnp.float32)
        for h in range(NH):
            rows = rows + d[h][pos2[k]].astype(jnp.float32)
        out = out + topk_weights[:, k][:, None] * rows
    return out
